# baseline (speedup 1.0000x reference)
import functools

import jax
import jax.numpy as jnp
from jax import lax
from jax.experimental import pallas as pl
from jax.experimental.pallas import tpu as pltpu
from jax.experimental.pallas import tpu_sc as plsc

B = 16384
D = 64
N = 1000000
NUM_CORES = 2
NUM_SUBCORES = 16
NW = NUM_CORES * NUM_SUBCORES
B_PER_W = B // NW

_mesh = plsc.VectorSubcoreMesh(core_axis_name="c", subcore_axis_name="s")


@functools.partial(
    pl.kernel,
    mesh=_mesh,
    out_type=jax.ShapeDtypeStruct((B, 2 * D), jnp.float32),
    scratch_types=[
        pltpu.VMEM((B_PER_W,), jnp.int32),
        pltpu.VMEM((B_PER_W,), jnp.int32),
        pltpu.VMEM((B_PER_W, 2 * D), jnp.float32),
        pltpu.SemaphoreType.DMA,
    ],
)
def _gather_kernel(idx_hbm, pairs_hbm, out_hbm, idx_v, ip_v, buf, sem):
    wid = lax.axis_index("s") * NUM_CORES + lax.axis_index("c")
    base = wid * B_PER_W
    pltpu.sync_copy(idx_hbm.at[pl.ds(base, B_PER_W)], idx_v)
    ip_v[...] = jnp.right_shift(idx_v[...], 1)
    pltpu.async_copy(pairs_hbm.at[ip_v], buf, sem).wait()
    pltpu.sync_copy(buf, out_hbm.at[pl.ds(base, B_PER_W)])


def kernel(idx, codes):
    idx_flat = idx.reshape(B).astype(jnp.int32)
    pairs = codes.reshape(N // 2, 2 * D)
    pairs_out = _gather_kernel(idx_flat, pairs)
    odd = (idx_flat % 2 == 1)[:, None]
    return jnp.where(odd, pairs_out[:, D:], pairs_out[:, :D])


# speedup vs baseline: 1.4877x; 1.4877x over previous
import functools

import jax
import jax.numpy as jnp
from jax import lax
from jax.experimental import pallas as pl
from jax.experimental.pallas import tpu as pltpu
from jax.experimental.pallas import tpu_sc as plsc

B = 16384
D = 64

NUM_CORES = 2
NUM_SUBCORES = 16
NW = NUM_CORES * NUM_SUBCORES
B_PER_W = B // NW
ROUND = 64
N_ROUNDS = B_PER_W // ROUND

_mesh = plsc.VectorSubcoreMesh(core_axis_name="c", subcore_axis_name="s")


@functools.partial(
    pl.kernel,
    mesh=_mesh,
    out_type=jax.ShapeDtypeStruct((B * 8, D), jnp.float32),
    scratch_types=[
        pltpu.VMEM((B_PER_W,), jnp.int32),
        pltpu.VMEM((ROUND * 8, D), jnp.float32),
        pltpu.SemaphoreType.DMA,
    ],
)
def _gather_kernel(idx_hbm, codes_hbm, grp_hbm, idx_v, grp, sem):
    wid = lax.axis_index("s") * NUM_CORES + lax.axis_index("c")
    base = wid * B_PER_W
    pltpu.sync_copy(idx_hbm.at[pl.ds(base, B_PER_W)], idx_v)

    def round_(t, _):
        def issue(c, _):
            vec = idx_v[pl.ds(t * ROUND + c * 16, 16)]
            gvec = jnp.left_shift(jnp.right_shift(vec, 3), 3)
            for j in range(16):
                g = pl.multiple_of(gvec[j], 8)
                pltpu.make_async_copy(
                    codes_hbm.at[pl.ds(g, 8), :],
                    grp.at[pl.ds((c * 16 + j) * 8, 8), :],
                    sem,
                ).start()
            return 0

        lax.fori_loop(0, ROUND // 16, issue, 0)
        pltpu.make_async_copy(
            codes_hbm.at[pl.ds(0, ROUND * 8), :], grp, sem
        ).wait()
        pltpu.sync_copy(
            grp, grp_hbm.at[pl.ds((base + t * ROUND) * 8, ROUND * 8)]
        )
        return 0

    lax.fori_loop(0, N_ROUNDS, round_, 0)


def kernel(idx, codes):
    idx_flat = idx.reshape(B).astype(jnp.int32)
    groups = _gather_kernel(idx_flat, codes)
    g = groups.reshape(B, 8, D)
    sub = (idx_flat & 7).astype(jnp.int32)
    return jnp.take_along_axis(g, sub[:, None, None], axis=1)[:, 0, :]
